# Initial kernel scaffold; baseline (speedup 1.0000x reference)
import jax
import jax.numpy as jnp
from jax.experimental import pallas as pl


def kernel(x, edge_index, mask, W, W_out, b_out):
    raise NotImplementedError("write your pallas kernel here")


# speedup vs baseline: 12.7751x; 12.7751x over previous
import functools

import jax
import jax.numpy as jnp
from jax import lax
from jax.experimental import pallas as pl
from jax.experimental.pallas import tpu as pltpu
from jax.experimental.pallas import tpu_sc as plsc

N = 10000
E = 320000
D = 128
T = 2

NC = 2
NS = 16
LANES = 128
K = 80
EPT = K * LANES
NVEC = EPT // 16
E_PAD = NC * NS * EPT
N_PAD = 10240
ROWS_PER_SUB = N_PAD // NS
HR = N_PAD // 128
BN = 2000
GRID = N // BN


def _deg_body(dst_hbm, zeros_hbm, iota_hbm, deg_out,
              idx_v, hist, col_v, i80_v, acc):
    cid = lax.axis_index("c")
    sid = lax.axis_index("s")
    pltpu.sync_copy(zeros_hbm, hist)

    @pl.when(sid == 0)
    def _():
        pltpu.sync_copy(hist, acc)

    plsc.subcore_barrier()
    pltpu.sync_copy(dst_hbm.at[cid, sid], idx_v)
    pltpu.sync_copy(iota_hbm, i80_v)

    @pl.loop(0, NVEC)
    def _(j):
        d = idx_v[j]
        cnt, last = plsc.scan_count(d)
        r = lax.shift_right_logical(d, 7)
        c = lax.bitwise_and(d, 127)
        plsc.addupdate_scatter(hist, [r, c], cnt.astype(jnp.float32),
                               mask=last)

    pltpu.sync_copy(hist, acc.at[i80_v], add=True)
    plsc.subcore_barrier()
    pltpu.sync_copy(acc.at[pl.ds(sid * 5, 5)], hist.at[pl.ds(0, 5)])
    for kk in range(ROWS_PER_SUB // 16):
        col_v[pl.ds(kk * 16, 16)] = hist[kk // 8, pl.ds((kk % 8) * 16, 16)]
    pltpu.sync_copy(col_v, deg_out.at[cid, pl.ds(sid * ROWS_PER_SUB,
                                                 ROWS_PER_SUB)])


def _agg_body(y_hbm, src_hbm, dst_hbm, zeros_hbm, agg_out,
              src_v, dst_v, rows_v, sem, acc):
    cid = lax.axis_index("c")
    sid = lax.axis_index("s")

    pltpu.sync_copy(zeros_hbm, rows_v)
    for t in range(ROWS_PER_SUB // LANES):
        pltpu.sync_copy(
            rows_v, acc.at[pl.ds(sid * ROWS_PER_SUB + t * LANES, LANES)])
    plsc.subcore_barrier()
    pltpu.sync_copy(src_hbm.at[cid, sid], src_v)
    pltpu.sync_copy(dst_hbm.at[cid, sid], dst_v)

    @pl.loop(0, K)
    def _(j):
        pltpu.async_copy(y_hbm.at[src_v.at[j]], rows_v, sem).wait()
        pltpu.sync_copy(rows_v, acc.at[dst_v.at[j]], add=True)

    plsc.subcore_barrier()
    for t in range(ROWS_PER_SUB // LANES):
        base = sid * ROWS_PER_SUB + t * LANES
        pltpu.sync_copy(acc.at[pl.ds(base, LANES)], rows_v)
        pltpu.sync_copy(rows_v, agg_out.at[cid, pl.ds(base, LANES)])


@functools.cache
def _sc_kernels():
    mesh = plsc.VectorSubcoreMesh(core_axis_name="c", subcore_axis_name="s")
    deg_kernel = pl.kernel(
        _deg_body,
        out_type=jax.ShapeDtypeStruct((NC, N_PAD), jnp.float32),
        mesh=mesh,
        compiler_params=pltpu.CompilerParams(needs_layout_passes=False),
        scratch_types=[
            pltpu.VMEM((NVEC, 16), jnp.int32),
            pltpu.VMEM((HR, 128), jnp.float32),
            pltpu.VMEM((ROWS_PER_SUB,), jnp.float32),
            pltpu.VMEM((HR,), jnp.int32),
            pltpu.VMEM_SHARED((HR, 128), jnp.float32),
        ],
    )
    agg_kernel = pl.kernel(
        _agg_body,
        out_type=jax.ShapeDtypeStruct((NC, N_PAD, D), jnp.float32),
        mesh=mesh,
        scratch_types=[
            pltpu.VMEM((K, LANES), jnp.int32),
            pltpu.VMEM((K, LANES), jnp.int32),
            pltpu.VMEM((LANES, D), jnp.float32),
            pltpu.SemaphoreType.DMA,
            pltpu.VMEM_SHARED((N_PAD, D), jnp.float32),
        ],
    )
    return deg_kernel, agg_kernel


def _y_body(x_ref, w_ref, degp_ref, y_ref):
    deg = degp_ref[0] + degp_ref[1] + 1.0
    dis = lax.rsqrt(deg)
    xw = jnp.dot(x_ref[...], w_ref[...], preferred_element_type=jnp.float32)
    y_ref[...] = xw * dis


def _fin_body(aggp_ref, y_ref, degp_ref, wout_ref, bout_ref, out_ref, h_ref):
    deg = degp_ref[0] + degp_ref[1] + 1.0
    dis = lax.rsqrt(deg)
    s = aggp_ref[0] + aggp_ref[1] + y_ref[...]
    h = jnp.maximum(dis * s, 0.0)
    h_ref[...] = h
    out_ref[...] = (
        jnp.dot(h, wout_ref[...], preferred_element_type=jnp.float32)
        + bout_ref[...])


def kernel(x, edge_index, mask, W, W_out, b_out):
    del mask
    src = edge_index[0]
    dst = edge_index[1]
    pad = E_PAD - E
    src_p = jnp.concatenate(
        [src, jnp.zeros((pad,), jnp.int32)]).reshape(NC, NS, K, LANES)
    dst_flat = jnp.concatenate([dst, jnp.full((pad,), N, jnp.int32)])
    dst_p = dst_flat.reshape(NC, NS, K, LANES)
    dst_p16 = dst_flat.reshape(NC, NS, NVEC, 16)

    zerosH = jnp.zeros((HR, 128), jnp.float32)
    zerosD = jnp.zeros((LANES, D), jnp.float32)
    iotaH = jnp.arange(HR, dtype=jnp.int32)

    deg_kernel, agg_kernel = _sc_kernels()
    degp = deg_kernel(dst_p16, zerosH, iotaH)
    degp3 = degp.reshape(NC, N_PAD, 1)

    y = pl.pallas_call(
        _y_body,
        grid=(GRID,),
        in_specs=[
            pl.BlockSpec((BN, D), lambda i: (i, 0)),
            pl.BlockSpec((D, D), lambda i: (0, 0)),
            pl.BlockSpec((NC, BN, 1), lambda i: (0, i, 0)),
        ],
        out_specs=pl.BlockSpec((BN, D), lambda i: (i, 0)),
        out_shape=jax.ShapeDtypeStruct((N, D), jnp.float32),
    )(x, W, degp3)

    aggp = agg_kernel(y, src_p, dst_p, zerosD)

    out, h = pl.pallas_call(
        _fin_body,
        grid=(GRID,),
        in_specs=[
            pl.BlockSpec((NC, BN, D), lambda i: (0, i, 0)),
            pl.BlockSpec((BN, D), lambda i: (i, 0)),
            pl.BlockSpec((NC, BN, 1), lambda i: (0, i, 0)),
            pl.BlockSpec((D, T), lambda i: (0, 0)),
            pl.BlockSpec((1, T), lambda i: (0, 0)),
        ],
        out_specs=[
            pl.BlockSpec((BN, T), lambda i: (i, 0)),
            pl.BlockSpec((BN, D), lambda i: (i, 0)),
        ],
        out_shape=[
            jax.ShapeDtypeStruct((N, T), jnp.float32),
            jax.ShapeDtypeStruct((N, D), jnp.float32),
        ],
    )(aggp, y, degp3, W_out, b_out.reshape(1, T))

    return (out, h)
